# baseline (speedup 1.0000x reference)
import functools

import jax
import jax.numpy as jnp
from jax import lax
from jax.experimental import pallas as pl
from jax.experimental.pallas import tpu as pltpu
from jax.experimental.pallas import tpu_sc as plsc


def kernel(seq, pos_table):
    B, L = seq.shape
    D = pos_table.shape[1]
    N = L * D
    tmpl = jax.lax.slice(pos_table, (1, 0), (L + 1, D)).reshape(N)
    row0 = jax.lax.slice(pos_table, (0, 0), (1, D)).reshape(D)
    seqf = seq.reshape(B * L)

    info = plsc.get_sparse_core_info()
    NC, NS = info.num_cores, info.num_subcores
    NW = NC * NS
    b_per_w = B // NW
    toks_w = b_per_w * L

    mesh = plsc.VectorSubcoreMesh(core_axis_name="c", subcore_axis_name="s")

    @functools.partial(
        pl.kernel, mesh=mesh,
        out_type=jax.ShapeDtypeStruct((B * N,), jnp.float32),
        scratch_types=[
            pltpu.VMEM((N,), jnp.float32),
            pltpu.VMEM((D,), jnp.float32),
            pltpu.VMEM((toks_w,), jnp.int32),
        ],
    )
    def sc_write(tmpl_hbm, row0_hbm, seqf_hbm, out_hbm, tmpl_v, row0_v, seq_v):
        wid = lax.axis_index("s") * NC + lax.axis_index("c")
        tok0 = wid * toks_w
        pltpu.sync_copy(tmpl_hbm, tmpl_v)
        pltpu.sync_copy(row0_hbm, row0_v)
        pltpu.sync_copy(seqf_hbm.at[pl.ds(tok0, toks_w)], seq_v)

        def bulk(b, carry):
            off = (wid * b_per_w + b) * N
            pltpu.sync_copy(tmpl_v, out_hbm.at[pl.ds(off, N)])
            return carry

        lax.fori_loop(0, b_per_w, bulk, 0)


    out1 = sc_write(tmpl, row0, seqf)
    return out1.reshape(B, L, D)


# speedup vs baseline: 1.0171x; 1.0171x over previous
import functools

import jax
import jax.numpy as jnp
from jax import lax
from jax.experimental import pallas as pl
from jax.experimental.pallas import tpu as pltpu
from jax.experimental.pallas import tpu_sc as plsc

_REP = 4


def kernel(seq, pos_table):
    B, L = seq.shape
    D = pos_table.shape[1]
    N = L * D
    tmpl = jax.lax.slice(pos_table, (1, 0), (L + 1, D)).reshape(N)
    tmpl4 = jnp.tile(tmpl, _REP)

    info = plsc.get_sparse_core_info()
    NC, NS = info.num_cores, info.num_subcores
    NW = NC * NS
    b_per_w = B // NW
    n_chunks = b_per_w // _REP
    CH = _REP * N

    mesh = plsc.VectorSubcoreMesh(core_axis_name="c", subcore_axis_name="s")

    @functools.partial(
        pl.kernel, mesh=mesh,
        out_type=jax.ShapeDtypeStruct((B * N,), jnp.float32),
        scratch_types=[
            pltpu.VMEM((CH,), jnp.float32),
            pltpu.SemaphoreType.DMA,
        ],
    )
    def sc_write(tmpl4_hbm, out_hbm, tmpl4_v, sem):
        wid = lax.axis_index("s") * NC + lax.axis_index("c")
        base = wid * b_per_w * N
        pltpu.sync_copy(tmpl4_hbm, tmpl4_v)

        def fire(c, carry):
            pltpu.async_copy(tmpl4_v, out_hbm.at[pl.ds(base + c * CH, CH)], sem)
            return carry

        lax.fori_loop(0, n_chunks, fire, 0)

        def drain(c, carry):
            pltpu.make_async_copy(
                tmpl4_v, out_hbm.at[pl.ds(base + c * CH, CH)], sem).wait()
            return carry

        lax.fori_loop(0, n_chunks, drain, 0)

    out1 = sc_write(tmpl4)
    return out1.reshape(B, L, D)


# speedup vs baseline: 2.2721x; 2.2339x over previous
import functools

import jax
import jax.numpy as jnp
import numpy as np
from jax.experimental import pallas as pl

_BLOCK_B = 128


def _body(seq_ref, e_ref, diff_ref, row0_ref, out_ref):
    m = (seq_ref[...] != 0).astype(jnp.bfloat16)
    maskex = jax.lax.dot_general(
        m, e_ref[...], (((1,), (0,)), ((), ())),
        preferred_element_type=jnp.float32)
    out_ref[...] = row0_ref[...] + maskex * diff_ref[...]


@functools.lru_cache(maxsize=None)
def _expansion_const(L, D):
    e = np.zeros((L, L * D), dtype=np.float32)
    for j in range(L):
        e[j, j * D:(j + 1) * D] = 1.0
    return e


def kernel(seq, pos_table):
    B, L = seq.shape
    D = pos_table.shape[1]
    N = L * D
    rows = jax.lax.slice(pos_table, (1, 0), (L + 1, D))
    row0 = jax.lax.slice(pos_table, (0, 0), (1, D))
    e = jnp.asarray(_expansion_const(L, D), dtype=jnp.bfloat16)
    row0t = jnp.tile(row0, (1, L))
    diff = rows.reshape(1, N) - row0t
    out2d = pl.pallas_call(
        _body,
        grid=(B // _BLOCK_B,),
        in_specs=[
            pl.BlockSpec((_BLOCK_B, L), lambda i: (i, 0)),
            pl.BlockSpec((L, N), lambda i: (0, 0)),
            pl.BlockSpec((1, N), lambda i: (0, 0)),
            pl.BlockSpec((1, N), lambda i: (0, 0)),
        ],
        out_specs=pl.BlockSpec((_BLOCK_B, N), lambda i: (i, 0)),
        out_shape=jax.ShapeDtypeStruct((B, N), pos_table.dtype),
    )(seq, e, diff, row0t)
    return out2d.reshape(B, L, D)


# speedup vs baseline: 2.2746x; 1.0011x over previous
import functools

import jax
import jax.numpy as jnp
import numpy as np
from jax import lax
from jax.experimental import pallas as pl
from jax.experimental.pallas import tpu as pltpu

_BLOCK_B = 128
_NBUF = 4


def _compute(seq_ref, e_ref, diff_ref, row0_ref):
    m = (seq_ref[...] != 0).astype(jnp.bfloat16)
    maskex = jax.lax.dot_general(
        m, e_ref[...], (((1,), (0,)), ((), ())),
        preferred_element_type=jnp.float32)
    return row0_ref[...] + maskex * diff_ref[...]


def _body(seq_ref, e_ref, diff_ref, row0_ref, out_ref, bufs, sems):
    i = pl.program_id(0)
    g = pl.num_programs(0)
    slot = lax.rem(i, _NBUF)

    @pl.when(i >= _NBUF)
    def _wait_prev():
        pltpu.make_async_copy(
            bufs.at[slot],
            out_ref.at[pl.ds((i - _NBUF) * _BLOCK_B, _BLOCK_B), :],
            sems.at[slot]).wait()

    bufs[slot] = _compute(seq_ref, e_ref, diff_ref, row0_ref)
    pltpu.make_async_copy(
        bufs.at[slot],
        out_ref.at[pl.ds(i * _BLOCK_B, _BLOCK_B), :],
        sems.at[slot]).start()

    @pl.when(i == g - 1)
    def _drain():
        for s in range(_NBUF):
            step = g - _NBUF + s
            pltpu.make_async_copy(
                bufs.at[lax.rem(step, _NBUF)],
                out_ref.at[pl.ds(step * _BLOCK_B, _BLOCK_B), :],
                sems.at[lax.rem(step, _NBUF)]).wait()


@functools.lru_cache(maxsize=None)
def _expansion_const(L, D):
    e = np.zeros((L, L * D), dtype=np.float32)
    for j in range(L):
        e[j, j * D:(j + 1) * D] = 1.0
    return e


def kernel(seq, pos_table):
    B, L = seq.shape
    D = pos_table.shape[1]
    N = L * D
    rows = jax.lax.slice(pos_table, (1, 0), (L + 1, D))
    row0 = jax.lax.slice(pos_table, (0, 0), (1, D))
    e = jnp.asarray(_expansion_const(L, D), dtype=jnp.bfloat16)
    row0t = jnp.tile(row0, (1, L))
    diff = rows.reshape(1, N) - row0t
    out2d = pl.pallas_call(
        _body,
        grid=(B // _BLOCK_B,),
        in_specs=[
            pl.BlockSpec((_BLOCK_B, L), lambda i: (i, 0)),
            pl.BlockSpec((L, N), lambda i: (0, 0)),
            pl.BlockSpec((1, N), lambda i: (0, 0)),
            pl.BlockSpec((1, N), lambda i: (0, 0)),
        ],
        out_specs=pl.BlockSpec(memory_space=pltpu.MemorySpace.HBM),
        out_shape=jax.ShapeDtypeStruct((B, N), pos_table.dtype),
        scratch_shapes=[
            pltpu.VMEM((_NBUF, _BLOCK_B, N), jnp.float32),
            pltpu.SemaphoreType.DMA((_NBUF,)),
        ],
    )(seq, e, diff, row0t)
    return out2d.reshape(B, L, D)


# speedup vs baseline: 2.2756x; 1.0004x over previous
import functools

import jax
import jax.numpy as jnp
import numpy as np
from jax.experimental import pallas as pl

_BLOCK_B = 128


def _body(seq_ref, e_ref, diff_ref, row0_ref, out_ref):
    m = (seq_ref[...] != 0).astype(jnp.bfloat16)
    maskex = jax.lax.dot_general(
        m, e_ref[...], (((1,), (0,)), ((), ())),
        preferred_element_type=jnp.float32)
    out_ref[...] = row0_ref[...] + maskex * diff_ref[...]


@functools.lru_cache(maxsize=None)
def _expansion_const(L, D):
    e = np.zeros((L, L * D), dtype=np.float32)
    for j in range(L):
        e[j, j * D:(j + 1) * D] = 1.0
    return e


def kernel(seq, pos_table):
    B, L = seq.shape
    D = pos_table.shape[1]
    N = L * D
    rows = jax.lax.slice(pos_table, (1, 0), (L + 1, D))
    row0 = jax.lax.slice(pos_table, (0, 0), (1, D))
    e = jnp.asarray(_expansion_const(L, D), dtype=jnp.bfloat16)
    row0t = jnp.tile(row0, (1, L))
    diff = rows.reshape(1, N) - row0t
    out2d = pl.pallas_call(
        _body,
        grid=(B // _BLOCK_B,),
        in_specs=[
            pl.BlockSpec((_BLOCK_B, L), lambda i: (i, 0)),
            pl.BlockSpec((L, N), lambda i: (0, 0)),
            pl.BlockSpec((1, N), lambda i: (0, 0)),
            pl.BlockSpec((1, N), lambda i: (0, 0)),
        ],
        out_specs=pl.BlockSpec((_BLOCK_B, N), lambda i: (i, 0)),
        out_shape=jax.ShapeDtypeStruct((B, N), pos_table.dtype),
    )(seq, e, diff, row0t)
    return out2d.reshape(B, L, D)
